# baseline (speedup 1.0000x reference)
import functools

import jax
import jax.numpy as jnp
from jax import lax
from jax.experimental import pallas as pl
from jax.experimental.pallas import tpu as pltpu
from jax.experimental.pallas import tpu_sc as plsc

NC, NS = 2, 16
NW = NC * NS
B, L, D = 1024, 200, 128
ROWS = B * L
RPW = ROWS // NW
C = 128
NCHUNK = RPW // C
NBUF = 4
UNROLL = 4
ADD = True

_mesh = plsc.VectorSubcoreMesh(
    core_axis_name="c", subcore_axis_name="s",
    num_cores=NC, num_subcores=NS)


@functools.partial(
    pl.kernel,
    out_type=jax.ShapeDtypeStruct((ROWS, D), jnp.float32),
    mesh=_mesh,
    scratch_types=(
        [pltpu.VMEM((RPW,), jnp.int32),
         pltpu.VMEM((L, D), jnp.float32)]
        + [pltpu.VMEM((C, D), jnp.float32) for _ in range(NBUF)]
        + [pltpu.SemaphoreType.DMA for _ in range(2 * NBUF + 1)]
    ),
)
def _embed(x_hbm, tok_hbm, pos_hbm, out_hbm, idx_v, pos_v, *scratch):
    bufs = scratch[:NBUF]
    gsems = scratch[NBUF:2 * NBUF]
    osems = scratch[2 * NBUF:3 * NBUF]
    psem = scratch[3 * NBUF]

    wid = lax.axis_index("s") * NC + lax.axis_index("c")
    base = wid * RPW

    pos_copy = pltpu.async_copy(pos_hbm, pos_v, psem)
    pltpu.sync_copy(x_hbm.at[pl.ds(base, RPW)], idx_v)

    def g_start(c, b):
        pltpu.async_copy(
            tok_hbm.at[idx_v.at[pl.ds(c * C, C)]], bufs[b], gsems[b])

    def g_wait(c, b):
        pltpu.make_async_copy(
            tok_hbm.at[idx_v.at[pl.ds(c * C, C)]], bufs[b], gsems[b]).wait()

    def o_start(c, b):
        pltpu.async_copy(
            bufs[b], out_hbm.at[pl.ds(base + c * C, C)], osems[b])

    def o_wait(c, b):
        pltpu.make_async_copy(
            bufs[b], out_hbm.at[pl.ds(base + c * C, C)], osems[b]).wait()

    def add_pos(c, b):
        buf = bufs[b]
        flat0 = c * C

        @plsc.parallel_loop(0, C, unroll=UNROLL)
        def _(r):
            pr = lax.rem(flat0 + r, L)
            for d8 in range(D // 16):
                plsc.addupdate(buf.at[r, pl.ds(d8 * 16, 16)],
                               pos_v[pr, pl.ds(d8 * 16, 16)])

    def step(c, b):
        g_wait(c, b)
        if ADD:
            add_pos(c, b)
        o_start(c, b)

    for c in range(NBUF - 1):
        g_start(c, c)

    pos_copy.wait()

    for c in range(NBUF):
        step(c, c % NBUF)
        if c >= 1:
            o_wait(c - 1, (c + NBUF - 1) % NBUF)
        g_start(c + NBUF - 1, (c + NBUF - 1) % NBUF)

    GMAX = (NCHUNK - NBUF) // NBUF
    @pl.loop(1, GMAX)
    def _(g):
        for b in range(NBUF):
            c = g * NBUF + b
            step(c, b)
            o_wait(c - 1, (b + NBUF - 1) % NBUF)
            g_start(c + NBUF - 1, (b + NBUF - 1) % NBUF)

    for c in range(GMAX * NBUF, NCHUNK):
        step(c, c % NBUF)
        if c + NBUF - 1 < NCHUNK:
            o_wait(c - 1, (c + NBUF - 1) % NBUF)
            g_start(c + NBUF - 1, (c + NBUF - 1) % NBUF)

    for c in range(NCHUNK - NBUF, NCHUNK):
        o_wait(c, c % NBUF)


def kernel(x, token_table, pos_table):
    out = _embed(x.reshape(ROWS), token_table, pos_table)
    return out.reshape(B, L, D)


# speedup vs baseline: 1.0231x; 1.0231x over previous
import functools

import jax
import jax.numpy as jnp
from jax import lax
from jax.experimental import pallas as pl
from jax.experimental.pallas import tpu as pltpu
from jax.experimental.pallas import tpu_sc as plsc

NC, NS = 2, 16
NW = NC * NS
B, L, D = 1024, 200, 128
ROWS = B * L
RPW = ROWS // NW
C = 128
NCHUNK = RPW // C
NBUF = 4
UNROLL = 2

_mesh = plsc.VectorSubcoreMesh(
    core_axis_name="c", subcore_axis_name="s",
    num_cores=NC, num_subcores=NS)


@functools.partial(
    pl.kernel,
    out_type=jax.ShapeDtypeStruct((ROWS, D), jnp.float32),
    mesh=_mesh,
    scratch_types=(
        [pltpu.VMEM((RPW,), jnp.int32),
         pltpu.VMEM((L, D), jnp.float32)]
        + [pltpu.VMEM((C, D), jnp.float32) for _ in range(NBUF)]
        + [pltpu.SemaphoreType.DMA for _ in range(2 * NBUF + 1)]
    ),
)
def _embed(x_hbm, tok_hbm, pos_hbm, out_hbm, idx_v, pos_v, *scratch):
    bufs = scratch[:NBUF]
    gsems = scratch[NBUF:2 * NBUF]
    osems = scratch[2 * NBUF:3 * NBUF]
    psem = scratch[3 * NBUF]

    wid = lax.axis_index("s") * NC + lax.axis_index("c")
    base = wid * RPW

    pos_copy = pltpu.async_copy(pos_hbm, pos_v, psem)
    pltpu.sync_copy(x_hbm.at[pl.ds(base, RPW)], idx_v)

    def g_start(c, b):
        pltpu.async_copy(
            tok_hbm.at[idx_v.at[pl.ds(c * C, C)]], bufs[b], gsems[b])

    def g_wait(c, b):
        pltpu.make_async_copy(
            tok_hbm.at[idx_v.at[pl.ds(c * C, C)]], bufs[b], gsems[b]).wait()

    def o_start(c, b):
        pltpu.async_copy(
            bufs[b], out_hbm.at[pl.ds(base + c * C, C)], osems[b])

    def o_wait(c, b):
        pltpu.make_async_copy(
            bufs[b], out_hbm.at[pl.ds(base + c * C, C)], osems[b]).wait()

    def add_pos(c, b):
        buf = bufs[b]
        flat0 = c * C

        @plsc.parallel_loop(0, C, unroll=UNROLL)
        def _(r):
            pr = lax.rem(flat0 + r, L)
            for d8 in range(D // 16):
                plsc.addupdate(buf.at[r, pl.ds(d8 * 16, 16)],
                               pos_v[pr, pl.ds(d8 * 16, 16)])

    def step(c, b):
        g_wait(c, b)
        add_pos(c, b)
        o_start(c, b)

    for c in range(NBUF - 1):
        g_start(c, c)

    pos_copy.wait()

    for c in range(NBUF):
        step(c, c % NBUF)
        if c >= 1:
            o_wait(c - 1, (c + NBUF - 1) % NBUF)
        g_start(c + NBUF - 1, (c + NBUF - 1) % NBUF)

    GMAX = (NCHUNK - NBUF) // NBUF
    @pl.loop(1, GMAX)
    def _(g):
        for b in range(NBUF):
            c = g * NBUF + b
            step(c, b)
            o_wait(c - 1, (b + NBUF - 1) % NBUF)
            g_start(c + NBUF - 1, (b + NBUF - 1) % NBUF)

    for c in range(GMAX * NBUF, NCHUNK):
        step(c, c % NBUF)
        if c + NBUF - 1 < NCHUNK:
            o_wait(c - 1, (c + NBUF - 1) % NBUF)
            g_start(c + NBUF - 1, (c + NBUF - 1) % NBUF)

    for c in range(NCHUNK - NBUF, NCHUNK):
        o_wait(c, c % NBUF)


def kernel(x, token_table, pos_table):
    out = _embed(x.reshape(ROWS), token_table, pos_table)
    return out.reshape(B, L, D)
